# Initial kernel scaffold; baseline (speedup 1.0000x reference)
import jax
import jax.numpy as jnp
from jax.experimental import pallas as pl


def kernel(node_features, edge_index, W_msg, b_msg, W_ih, b_ih, W_hh, b_hh, W_cls, b_cls):
    raise NotImplementedError("write your pallas kernel here")


# speedup vs baseline: 29.7328x; 29.7328x over previous
import functools

import jax
import jax.numpy as jnp
from jax import lax
from jax.experimental import pallas as pl
from jax.experimental.pallas import tpu as pltpu
from jax.experimental.pallas import tpu_sc as plsc

NC = 2
NS = 16
NW = NC * NS

K = 1024
KR = K // 128


def _sc_aggregate(table, src2d, dst2d, n, n_acc, ch):
    mesh = plsc.VectorSubcoreMesh(core_axis_name="c", subcore_axis_name="s")
    z_per_sub = n_acc // NS

    @functools.partial(
        pl.kernel,
        out_type=jax.ShapeDtypeStruct((NC, n_acc, 16), jnp.float32),
        mesh=mesh,
        scratch_types=[
            pltpu.VMEM_SHARED((n_acc, 16), jnp.float32),
            pltpu.VMEM((KR, 128), jnp.int32),
            pltpu.VMEM((KR, 128), jnp.int32),
            pltpu.VMEM((K, 16), jnp.float32),
            pltpu.SemaphoreType.DMA,
        ],
        compiler_params=pltpu.CompilerParams(use_tc_tiling_on_sc=False),
    )
    def kern(tab_hbm, src_hbm, dst_hbm, out_hbm, accum, sidx, didx, rows,
             sem):
        c = lax.axis_index("c")
        s = lax.axis_index("s")

        @pl.loop(0, K)
        def _(i):
            rows[i, :] = jnp.zeros((16,), jnp.float32)

        zbase = s * z_per_sub
        nfull, rem = divmod(z_per_sub, K)
        for q in range(nfull):
            pltpu.sync_copy(rows, accum.at[pl.ds(zbase + q * K, K)])
        if rem:
            pltpu.sync_copy(rows.at[pl.ds(0, rem)],
                            accum.at[pl.ds(zbase + nfull * K, rem)])
        plsc.subcore_barrier()

        w = c * NS + s
        rbase = w * (ch * KR)

        @pl.loop(0, ch)
        def _(j):
            rb = rbase + j * KR
            pltpu.sync_copy(src_hbm.at[pl.ds(rb, KR)], sidx)
            pltpu.sync_copy(dst_hbm.at[pl.ds(rb, KR)], didx)
            gs = [
                pltpu.async_copy(tab_hbm.at[sidx.at[jr]],
                                 rows.at[pl.ds(jr * 128, 128)], sem)
                for jr in range(KR)
            ]
            for d in gs:
                d.wait()
            ss = [
                pltpu.async_copy(rows.at[pl.ds(jr * 128, 128)],
                                 accum.at[didx.at[jr]], sem, add=True)
                for jr in range(KR)
            ]
            for d in ss:
                d.wait()

        plsc.subcore_barrier()

        ob = s * z_per_sub
        pltpu.sync_copy(accum.at[pl.ds(ob, z_per_sub)],
                        out_hbm.at[c, pl.ds(ob, z_per_sub)])

    return kern(table, src2d, dst2d)


def _dense_body(part_ref, nf_ref, cons_ref, wih_ref, wcls_ref, out_ref):
    acc = part_ref[0] + part_ref[1]
    cnt = acc[:, 2:3]
    inv = 1.0 / jnp.maximum(cnt, 1.0)
    has = cnt > 0.0
    a0 = jnp.where(has, acc[:, 0:1] * inv, nf_ref[:, 0:1])
    a1 = jnp.where(has, acc[:, 1:2] * inv, nf_ref[:, 1:2])

    cons = cons_ref[...]
    w0 = cons[2:3, 0:16]
    w1 = cons[3:4, 0:16]
    bm = cons[4:5, 0:16]
    x = a0 * w0 + a1 * w1 + bm

    gi = jax.lax.dot_general(
        x, wih_ref[...], (((1,), (0,)), ((), ())),
        preferred_element_type=jnp.float32) + cons[0:1, :]
    r = jax.nn.sigmoid(gi[:, 0:16])
    z = jax.nn.sigmoid(gi[:, 16:32])
    ng = jnp.tanh(gi[:, 32:48] + r * cons[1:2, 32:48])
    h = (1.0 - z) * ng

    logits = jax.lax.dot_general(
        h, wcls_ref[...], (((1,), (0,)), ((), ())),
        preferred_element_type=jnp.float32) + cons[5:6, 0:2]
    out_ref[...] = logits


def _tc_dense(partials, table, cons, wih_t, wcls_t, n):
    B = 2000
    return pl.pallas_call(
        _dense_body,
        out_shape=jax.ShapeDtypeStruct((n, 2), jnp.float32),
        grid=(n // B,),
        in_specs=[
            pl.BlockSpec((NC, B, 16), lambda i: (0, i, 0)),
            pl.BlockSpec((B, 16), lambda i: (i, 0)),
            pl.BlockSpec((8, 48), lambda i: (0, 0)),
            pl.BlockSpec((16, 48), lambda i: (0, 0)),
            pl.BlockSpec((16, 2), lambda i: (0, 0)),
        ],
        out_specs=pl.BlockSpec((B, 2), lambda i: (i, 0)),
    )(partials, table, cons, wih_t, wcls_t)


def kernel(node_features, edge_index, W_msg, b_msg, W_ih, b_ih, W_hh, b_hh,
           W_cls, b_cls):
    n = node_features.shape[0]
    e = edge_index.shape[1]

    table = jnp.concatenate(
        [node_features,
         jnp.ones((n, 1), jnp.float32),
         jnp.zeros((n, 13), jnp.float32)], axis=1)
    table_pad = jnp.concatenate([table, jnp.zeros((8, 16), jnp.float32)],
                                axis=0)

    per_w = -(-e // (NW * K)) * K
    ch = per_w // K
    e_pad = per_w * NW
    src = jnp.concatenate(
        [edge_index[0], jnp.full((e_pad - e,), n, jnp.int32)]).reshape(-1, 128)
    dst = jnp.concatenate(
        [edge_index[1], jnp.full((e_pad - e,), n, jnp.int32)]).reshape(-1, 128)

    n_acc = NS * 8 * (-(-(n + 8) // (NS * 8)))

    partials = _sc_aggregate(table_pad, src, dst, n, n_acc, ch)

    bgate = jnp.concatenate([b_ih[0:32] + b_hh[0:32], b_ih[32:48]])
    row = lambda v: jnp.pad(v, (0, 48 - v.shape[0]))[None, :]
    cons = jnp.concatenate([
        row(bgate),
        row(b_hh),
        row(W_msg[:, 0]),
        row(W_msg[:, 1]),
        row(b_msg),
        row(b_cls),
        jnp.zeros((2, 48), jnp.float32),
    ], axis=0)

    return _tc_dense(partials, table, cons, W_ih.T, W_cls.T, n)


# speedup vs baseline: 47.3924x; 1.5939x over previous
import functools

import jax
import jax.numpy as jnp
from jax import lax
from jax.experimental import pallas as pl
from jax.experimental.pallas import tpu as pltpu
from jax.experimental.pallas import tpu_sc as plsc

NC = 2
NS = 16
NW = NC * NS

K = 1024
KR = K // 128
W = 8


def _sc_aggregate(table, zeros4, src2d, dst2d, n_acc, ch):
    mesh = plsc.VectorSubcoreMesh(core_axis_name="c", subcore_axis_name="s")
    z_per_sub = n_acc // NS

    @functools.partial(
        pl.kernel,
        out_type=jax.ShapeDtypeStruct((NC, n_acc, W), jnp.float32),
        mesh=mesh,
        scratch_types=[
            pltpu.VMEM_SHARED((n_acc, W), jnp.float32),
            pltpu.VMEM_SHARED((n_acc, W), jnp.float32),
            pltpu.VMEM((2, KR, 128), jnp.int32),
            pltpu.VMEM((2, KR, 128), jnp.int32),
            pltpu.VMEM((2, K, W), jnp.float32),
            pltpu.SemaphoreType.DMA,
            pltpu.SemaphoreType.DMA,
            pltpu.SemaphoreType.DMA,
        ],
        compiler_params=pltpu.CompilerParams(use_tc_tiling_on_sc=False),
    )
    def kern(tab_hbm, z_hbm, src_hbm, dst_hbm, out_hbm, tab, accum, sidx,
             didx, rows, gsem, ssem, isem):
        c = lax.axis_index("c")
        s = lax.axis_index("s")

        zbase = s * z_per_sub
        pltpu.async_copy(tab_hbm.at[pl.ds(zbase, z_per_sub)],
                         tab.at[pl.ds(zbase, z_per_sub)], gsem)
        pltpu.async_copy(z_hbm.at[pl.ds(zbase, z_per_sub)],
                         accum.at[pl.ds(zbase, z_per_sub)], ssem)

        w = c * NS + s
        rbase = w * (ch * KR)
        pltpu.async_copy(src_hbm.at[pl.ds(rbase, KR)], sidx.at[0], isem)
        pltpu.async_copy(dst_hbm.at[pl.ds(rbase, KR)], didx.at[0], isem)

        pltpu.make_async_copy(tab_hbm.at[pl.ds(zbase, z_per_sub)],
                              tab.at[pl.ds(zbase, z_per_sub)], gsem).wait()
        pltpu.make_async_copy(z_hbm.at[pl.ds(zbase, z_per_sub)],
                              accum.at[pl.ds(zbase, z_per_sub)], ssem).wait()
        plsc.subcore_barrier()

        def drain_scatters(buf):
            for jr in range(KR):
                pltpu.make_async_copy(rows.at[buf, pl.ds(jr * 128, 128)],
                                      accum.at[didx.at[buf, jr]],
                                      ssem).wait()

        def chunk(j, buf, first, last):
            pltpu.make_async_copy(src_hbm.at[pl.ds(rbase, KR)],
                                  sidx.at[buf], isem).wait()
            pltpu.make_async_copy(dst_hbm.at[pl.ds(rbase, KR)],
                                  didx.at[buf], isem).wait()
            gs = [
                pltpu.async_copy(tab.at[sidx.at[buf, jr]],
                                 rows.at[buf, pl.ds(jr * 128, 128)], gsem)
                for jr in range(KR)
            ]
            if not first:
                drain_scatters(1 - buf)
            if not last:
                rb = rbase + (j + 1) * KR
                pltpu.async_copy(src_hbm.at[pl.ds(rb, KR)],
                                 sidx.at[1 - buf], isem)
                pltpu.async_copy(dst_hbm.at[pl.ds(rb, KR)],
                                 didx.at[1 - buf], isem)
            for d in gs:
                d.wait()
            for jr in range(KR):
                pltpu.async_copy(rows.at[buf, pl.ds(jr * 128, 128)],
                                 accum.at[didx.at[buf, jr]], ssem, add=True)

        chunk(0, 0, True, False)

        @pl.loop(0, (ch - 2) // 2)
        def _(jj):
            chunk(2 * jj + 1, 1, False, False)
            chunk(2 * jj + 2, 0, False, False)

        chunk(ch - 1, 1, False, True)
        drain_scatters(1)

        plsc.subcore_barrier()

        pltpu.sync_copy(accum.at[pl.ds(zbase, z_per_sub)],
                        out_hbm.at[c, pl.ds(zbase, z_per_sub)])

    return kern(table, zeros4, src2d, dst2d)


def _dense_body(part_ref, nf_ref, cons_ref, wih_ref, wcls_ref, out_ref):
    acc = part_ref[0] + part_ref[1]
    cnt = acc[:, 2:3]
    inv = 1.0 / jnp.maximum(cnt, 1.0)
    has = cnt > 0.0
    a0 = jnp.where(has, acc[:, 0:1] * inv, nf_ref[:, 0:1])
    a1 = jnp.where(has, acc[:, 1:2] * inv, nf_ref[:, 1:2])

    cons = cons_ref[...]
    w0 = cons[2:3, 0:16]
    w1 = cons[3:4, 0:16]
    bm = cons[4:5, 0:16]
    x = a0 * w0 + a1 * w1 + bm

    gi = jax.lax.dot_general(
        x, wih_ref[...], (((1,), (0,)), ((), ())),
        preferred_element_type=jnp.float32) + cons[0:1, :]
    r = jax.nn.sigmoid(gi[:, 0:16])
    z = jax.nn.sigmoid(gi[:, 16:32])
    ng = jnp.tanh(gi[:, 32:48] + r * cons[1:2, 32:48])
    h = (1.0 - z) * ng

    logits = jax.lax.dot_general(
        h, wcls_ref[...], (((1,), (0,)), ((), ())),
        preferred_element_type=jnp.float32) + cons[5:6, 0:2]
    out_ref[...] = logits


def _tc_dense(partials, table, cons, wih_t, wcls_t, n):
    B = 2000
    return pl.pallas_call(
        _dense_body,
        out_shape=jax.ShapeDtypeStruct((n, 2), jnp.float32),
        grid=(n // B,),
        in_specs=[
            pl.BlockSpec((NC, B, W), lambda i: (0, i, 0)),
            pl.BlockSpec((B, W), lambda i: (i, 0)),
            pl.BlockSpec((8, 48), lambda i: (0, 0)),
            pl.BlockSpec((16, 48), lambda i: (0, 0)),
            pl.BlockSpec((16, 2), lambda i: (0, 0)),
        ],
        out_specs=pl.BlockSpec((B, 2), lambda i: (i, 0)),
    )(partials, table, cons, wih_t, wcls_t)


def kernel(node_features, edge_index, W_msg, b_msg, W_ih, b_ih, W_hh, b_hh,
           W_cls, b_cls):
    n = node_features.shape[0]
    e = edge_index.shape[1]

    n_acc = NS * 8 * (-(-(n + 8) // (NS * 8)))

    table = jnp.concatenate(
        [node_features, jnp.ones((n, 1), jnp.float32)], axis=1)
    table = jnp.zeros((n_acc, W), jnp.float32).at[:n, :3].set(table)
    zeros4 = jnp.zeros((n_acc, W), jnp.float32)

    per_w = -(-e // (NW * 2 * K)) * 2 * K
    ch = per_w // K
    e_pad = per_w * NW
    src = jnp.concatenate(
        [edge_index[0], jnp.full((e_pad - e,), n, jnp.int32)]).reshape(-1, 128)
    dst = jnp.concatenate(
        [edge_index[1], jnp.full((e_pad - e,), n, jnp.int32)]).reshape(-1, 128)

    partials = _sc_aggregate(table, zeros4, src, dst, n_acc, ch)

    bgate = jnp.concatenate([b_ih[0:32] + b_hh[0:32], b_ih[32:48]])
    row = lambda v: jnp.pad(v, (0, 48 - v.shape[0]))[None, :]
    cons = jnp.concatenate([
        row(bgate),
        row(b_hh),
        row(W_msg[:, 0]),
        row(W_msg[:, 1]),
        row(b_msg),
        row(b_cls),
        jnp.zeros((2, 48), jnp.float32),
    ], axis=0)

    return _tc_dense(partials, table, cons, W_ih.T, W_cls.T, n)


# speedup vs baseline: 54.1254x; 1.1421x over previous
import functools

import jax
import jax.numpy as jnp
from jax import lax
from jax.experimental import pallas as pl
from jax.experimental.pallas import tpu as pltpu
from jax.experimental.pallas import tpu_sc as plsc

NC = 2
NS = 16
NW = NC * NS

K = 1024
KR = K // 128
W = 8


def _sc_aggregate(table, zeros4, src2d, dst2d, n_acc, ch):
    mesh = plsc.VectorSubcoreMesh(core_axis_name="c", subcore_axis_name="s")
    z_per_sub = n_acc // NS

    @functools.partial(
        pl.kernel,
        out_type=jax.ShapeDtypeStruct((NC, n_acc, W), jnp.float32),
        mesh=mesh,
        scratch_types=[
            pltpu.VMEM_SHARED((n_acc, W), jnp.float32),
            pltpu.VMEM_SHARED((n_acc, W), jnp.float32),
            pltpu.VMEM((2, KR, 128), jnp.int32),
            pltpu.VMEM((2, KR, 128), jnp.int32),
            pltpu.VMEM((2, K, W), jnp.float32),
            pltpu.SemaphoreType.DMA,
            pltpu.SemaphoreType.DMA,
            pltpu.SemaphoreType.DMA,
        ],
        compiler_params=pltpu.CompilerParams(use_tc_tiling_on_sc=False),
    )
    def kern(tab_hbm, z_hbm, src_hbm, dst_hbm, out_hbm, tab, accum, sidx,
             didx, rows, gsem, ssem, isem):
        c = lax.axis_index("c")
        s = lax.axis_index("s")

        zbase = s * z_per_sub
        pltpu.async_copy(tab_hbm.at[pl.ds(zbase, z_per_sub)],
                         tab.at[pl.ds(zbase, z_per_sub)], gsem)
        pltpu.async_copy(z_hbm.at[pl.ds(zbase, z_per_sub)],
                         accum.at[pl.ds(zbase, z_per_sub)], ssem)

        w = c * NS + s
        rbase = w * (ch * KR)
        pltpu.async_copy(src_hbm.at[pl.ds(rbase, KR)], sidx.at[0], isem)
        pltpu.async_copy(dst_hbm.at[pl.ds(rbase, KR)], didx.at[0], isem)

        pltpu.make_async_copy(tab_hbm.at[pl.ds(zbase, z_per_sub)],
                              tab.at[pl.ds(zbase, z_per_sub)], gsem).wait()
        pltpu.make_async_copy(z_hbm.at[pl.ds(zbase, z_per_sub)],
                              accum.at[pl.ds(zbase, z_per_sub)], ssem).wait()
        plsc.subcore_barrier()

        def drain_scatters(buf):
            for jr in range(KR):
                pltpu.make_async_copy(rows.at[buf, pl.ds(jr * 128, 128)],
                                      accum.at[didx.at[buf, jr]],
                                      ssem).wait()

        def chunk(j, buf, first, last):
            pltpu.make_async_copy(src_hbm.at[pl.ds(rbase, KR)],
                                  sidx.at[buf], isem).wait()
            pltpu.make_async_copy(dst_hbm.at[pl.ds(rbase, KR)],
                                  didx.at[buf], isem).wait()
            gs = [
                pltpu.async_copy(tab.at[sidx.at[buf, jr]],
                                 rows.at[buf, pl.ds(jr * 128, 128)], gsem)
                for jr in range(KR)
            ]
            if not first:
                drain_scatters(1 - buf)
            if not last:
                rb = rbase + (j + 1) * KR
                pltpu.async_copy(src_hbm.at[pl.ds(rb, KR)],
                                 sidx.at[1 - buf], isem)
                pltpu.async_copy(dst_hbm.at[pl.ds(rb, KR)],
                                 didx.at[1 - buf], isem)
            for d in gs:
                d.wait()
            for jr in range(KR):
                pltpu.async_copy(rows.at[buf, pl.ds(jr * 128, 128)],
                                 accum.at[didx.at[buf, jr]], ssem, add=True)

        chunk(0, 0, True, False)

        @pl.loop(0, (ch - 2) // 2)
        def _(jj):
            chunk(2 * jj + 1, 1, False, False)
            chunk(2 * jj + 2, 0, False, False)

        chunk(ch - 1, 1, False, True)
        drain_scatters(1)

        plsc.subcore_barrier()

        pltpu.sync_copy(accum.at[pl.ds(zbase, z_per_sub)],
                        out_hbm.at[c, pl.ds(zbase, z_per_sub)])

    return kern(table, zeros4, src2d, dst2d)


H = 16


def _dense_body(part_ref, nf_ref, cons_ref, out_ref):
    c = cons_ref
    s0 = part_ref[0, 0] + part_ref[1, 0]
    s1 = part_ref[0, 1] + part_ref[1, 1]
    cnt = part_ref[0, 2] + part_ref[1, 2]
    inv = 1.0 / jnp.maximum(cnt, 1.0)
    has = cnt > 0.0
    a0 = jnp.where(has, s0 * inv, nf_ref[0])
    a1 = jnp.where(has, s1 * inv, nf_ref[1])

    x = [a0 * c[48, h] + a1 * c[49, h] + c[50, h] for h in range(H)]

    def gate(j):
        g = c[52 + j // 16, j % 16]
        for h in range(H):
            g = g + x[h] * c[j, h]
        return g

    hid = []
    for j in range(H):
        r = jax.nn.sigmoid(gate(j))
        z = jax.nn.sigmoid(gate(16 + j))
        ng = jnp.tanh(gate(32 + j) + r * c[51, j])
        hid.append((1.0 - z) * ng)

    for i in range(2):
        l = c[57, i]
        for j in range(H):
            l = l + hid[j] * c[55 + i, j]
        out_ref[i] = l


def _tc_dense(partials_t, table_t, cons, n_acc):
    nr = n_acc // 128
    return pl.pallas_call(
        _dense_body,
        out_shape=jax.ShapeDtypeStruct((2, nr, 128), jnp.float32),
        grid=(nr // 8,),
        in_specs=[
            pl.BlockSpec((NC, W, 8, 128), lambda i: (0, 0, i, 0)),
            pl.BlockSpec((2, 8, 128), lambda i: (0, i, 0)),
            pl.BlockSpec(memory_space=pltpu.SMEM),
        ],
        out_specs=pl.BlockSpec((2, 8, 128), lambda i: (0, i, 0)),
    )(partials_t, table_t, cons)


def kernel(node_features, edge_index, W_msg, b_msg, W_ih, b_ih, W_hh, b_hh,
           W_cls, b_cls):
    n = node_features.shape[0]
    e = edge_index.shape[1]

    n_acc = NS * 8 * (-(-(n + 8) // (NS * 8)))

    table = jnp.concatenate(
        [node_features, jnp.ones((n, 1), jnp.float32)], axis=1)
    table = jnp.zeros((n_acc, W), jnp.float32).at[:n, :3].set(table)
    zeros4 = jnp.zeros((n_acc, W), jnp.float32)

    per_w = -(-e // (NW * 2 * K)) * 2 * K
    ch = per_w // K
    e_pad = per_w * NW
    src = jnp.concatenate(
        [edge_index[0], jnp.full((e_pad - e,), n, jnp.int32)]).reshape(-1, 128)
    dst = jnp.concatenate(
        [edge_index[1], jnp.full((e_pad - e,), n, jnp.int32)]).reshape(-1, 128)

    partials = _sc_aggregate(table, zeros4, src, dst, n_acc, ch)

    nr = n_acc // 128
    partials_t = partials.transpose(0, 2, 1).reshape(NC, W, nr, 128)
    table_t = table[:, :2].T.reshape(2, nr, 128)

    bgate = jnp.concatenate([b_ih[0:32] + b_hh[0:32], b_ih[32:48]])
    cons = jnp.concatenate([
        W_ih,
        W_msg[:, 0][None, :],
        W_msg[:, 1][None, :],
        b_msg[None, :],
        b_hh[32:48][None, :],
        bgate.reshape(3, 16),
        W_cls,
        jnp.pad(b_cls, (0, 14))[None, :],
    ], axis=0)

    out_t = _tc_dense(partials_t, table_t, cons, n_acc)
    return out_t.reshape(2, n_acc)[:, :n].T
